# baseline (speedup 1.0000x reference)
import functools

import jax
import jax.numpy as jnp
from jax import lax
from jax.experimental import pallas as pl
from jax.experimental.pallas import tpu as pltpu
from jax.experimental.pallas import tpu_sc as plsc

N_ENT = 100000
N_REL = 100000
H_DIM = 64
BATCH = 16384
REG_PARAM = 0.01

NC = 2
NS = 16
NW = NC * NS
L = 16

PW = BATCH // NW
C = 256
NCHUNK = PW // C
G = C // L


def _sc_body(emb_hbm, wrel_hbm, wrinv_hbm, hidx_hbm, ridx_hbm, tidx_hbm,
             scores_hbm, sq_hbm,
             hidx_v, ridx_v, tidx_v, hrows, trows, rrows, irows,
             scores_v, sq_v, sem):
    wid = lax.axis_index("s") * NC + lax.axis_index("c")
    wbase = wid * PW

    sq_acc0 = jnp.zeros((L,), jnp.float32)

    def chunk(c, sq_acc):
        cbase = wbase + c * C
        pltpu.sync_copy(hidx_hbm.at[pl.ds(cbase, C)], hidx_v)
        pltpu.sync_copy(ridx_hbm.at[pl.ds(cbase, C)], ridx_v)
        pltpu.sync_copy(tidx_hbm.at[pl.ds(cbase, C)], tidx_v)

        cp_h = pltpu.async_copy(emb_hbm.at[hidx_v], hrows, sem)
        cp_t = pltpu.async_copy(emb_hbm.at[tidx_v], trows, sem)
        cp_r = pltpu.async_copy(wrel_hbm.at[ridx_v], rrows, sem)
        cp_i = pltpu.async_copy(wrinv_hbm.at[ridx_v], irows, sem)
        cp_h.wait()
        cp_t.wait()
        cp_r.wait()
        cp_i.wait()

        def group(g, sq):
            rows = g * L + lax.iota(jnp.int32, L)
            score = jnp.zeros((L,), jnp.float32)
            for j in range(H_DIM):
                col = jnp.full((L,), j, jnp.int32)
                vh = plsc.load_gather(hrows, [rows, col])
                vt = plsc.load_gather(trows, [rows, col])
                vr = plsc.load_gather(rrows, [rows, col])
                vi = plsc.load_gather(irows, [rows, col])
                score = score + vh * vt * (vr + vi)
                sq = sq + (vh * vh + vt * vt) + (vr * vr + vi * vi)
            scores_v[pl.ds(c * C + g * L, L)] = score * 0.5
            return sq

        return lax.fori_loop(0, G, group, sq_acc)

    sq_acc = sq_acc0
    for c in range(NCHUNK):
        sq_acc = chunk(c, sq_acc)

    sq_v[...] = sq_acc
    pltpu.sync_copy(scores_v, scores_hbm.at[pl.ds(wbase, PW)])
    pltpu.sync_copy(sq_v, sq_hbm.at[pl.ds(wid * L, L)])


@jax.jit
def _sc_gather_score(emb, wrel, wrinv, hidx, ridx, tidx):
    mesh = plsc.VectorSubcoreMesh(core_axis_name="c", subcore_axis_name="s")
    f = functools.partial(
        pl.kernel,
        out_type=[
            jax.ShapeDtypeStruct((BATCH,), jnp.float32),
            jax.ShapeDtypeStruct((NW * L,), jnp.float32),
        ],
        mesh=mesh,
        compiler_params=pltpu.CompilerParams(
            use_tc_tiling_on_sc=False, needs_layout_passes=False),
        scratch_types=[
            pltpu.VMEM((C,), jnp.int32),
            pltpu.VMEM((C,), jnp.int32),
            pltpu.VMEM((C,), jnp.int32),
            pltpu.VMEM((C, H_DIM), jnp.float32),
            pltpu.VMEM((C, H_DIM), jnp.float32),
            pltpu.VMEM((C, H_DIM), jnp.float32),
            pltpu.VMEM((C, H_DIM), jnp.float32),
            pltpu.VMEM((PW,), jnp.float32),
            pltpu.VMEM((L,), jnp.float32),
            pltpu.SemaphoreType.DMA,
        ],
    )(_sc_body)
    return f(emb, wrel, wrinv, hidx, ridx, tidx)


def _tc_loss_body(s_ref, l_ref, q_ref, o_ref):
    s = s_ref[...]
    lbl = l_ref[...]
    bce = jnp.maximum(s, 0.0) - s * lbl + jnp.log1p(jnp.exp(-jnp.abs(s)))
    predict_loss = jnp.sum(bce) / BATCH
    reg_loss = jnp.sum(q_ref[...]) / (4.0 * BATCH * H_DIM)
    o_ref[0, 0] = predict_loss + REG_PARAM * reg_loss


@jax.jit
def _tc_loss(scores, labels, sqsums):
    out = pl.pallas_call(
        _tc_loss_body,
        out_shape=jax.ShapeDtypeStruct((1, 1), jnp.float32),
        out_specs=pl.BlockSpec(memory_space=pltpu.SMEM),
    )(scores.reshape(128, 128), labels.reshape(128, 128),
      sqsums.reshape(4, 128))
    return out[0, 0]


def kernel(mixedEmbedding, w_relation, w_relation_inv, triplets, labels):
    hidx = triplets[:, 0]
    ridx = triplets[:, 1]
    tidx = triplets[:, 2]
    scores, sqsums = _sc_gather_score(
        mixedEmbedding, w_relation, w_relation_inv, hidx, ridx, tidx)
    return _tc_loss(scores, labels, sqsums)
